# Initial kernel scaffold; baseline (speedup 1.0000x reference)
import jax
import jax.numpy as jnp
from jax.experimental import pallas as pl


def kernel(x, edge_index, pos_edge_index, neg_edge_index, Wself1, Wneigh1, bs1, Wself2, Wneigh2, bs2, Wp1, bp1, Wp2, bp2):
    raise NotImplementedError("write your pallas kernel here")


# speedup vs baseline: 3.1373x; 3.1373x over previous
import functools

import jax
import jax.numpy as jnp
from jax import lax
from jax.experimental import pallas as pl
from jax.experimental.pallas import tpu as pltpu
from jax.experimental.pallas import tpu_sc as plsc

NC = 2
NS = 16
LANES = 16
NW = NC * NS
CH = 80
RCH = 80


def _make_seg_sum(n, d, e):
    e_per_w = e // NW
    n_iters = e_per_w // CH
    assert e_per_w * NW == e and n_iters * CH == e_per_w
    n_chunks = n // RCH
    assert n_chunks * RCH == n
    chunk_trips = -(-n_chunks // NS)
    kg = d // LANES

    mesh = plsc.VectorSubcoreMesh(
        core_axis_name="c", subcore_axis_name="s", num_cores=NC, num_subcores=NS
    )
    assert RCH == CH
    scratch = [
        pltpu.VMEM((n_iters, CH), jnp.int32),
        pltpu.VMEM((n_iters, CH), jnp.int32),
        pltpu.VMEM((CH, d), jnp.float32),
        pltpu.VMEM_SHARED((n, d), jnp.float32),
        pltpu.SemaphoreType.DMA,
    ]

    def body(x_hbm, src_hbm, dst_hbm, out_hbm,
             src_v, dst_v, rows_v, agg_sh, sem):
        c = lax.axis_index("c")
        s = lax.axis_index("s")
        wid = c * NS + s
        zero16 = jnp.zeros((LANES,), jnp.float32)

        def zb(r, _):
            for k in range(kg):
                rows_v[r, pl.ds(k * LANES, LANES)] = zero16
            return 0

        lax.fori_loop(0, RCH, zb, 0)

        def zs(ti, _):
            t = s + ti * NS

            @pl.when(t < n_chunks)
            def _():
                pltpu.sync_copy(rows_v, agg_sh.at[pl.ds(t * RCH, RCH)])

            return 0

        lax.fori_loop(0, chunk_trips, zs, 0)

        pltpu.sync_copy(src_hbm.at[wid], src_v)
        pltpu.sync_copy(dst_hbm.at[wid], dst_v)
        plsc.subcore_barrier()

        def step(j, _):
            pltpu.async_copy(x_hbm.at[src_v.at[j]], rows_v, sem).wait()
            pltpu.sync_copy(rows_v, agg_sh.at[dst_v.at[j]], add=True)
            return 0

        lax.fori_loop(0, n_iters, step, 0)
        plsc.subcore_barrier()

        def wr(ti, _):
            t = s + ti * NS

            @pl.when(t < n_chunks)
            def _():
                pltpu.sync_copy(
                    agg_sh.at[pl.ds(t * RCH, RCH)],
                    out_hbm.at[c, pl.ds(t * RCH, RCH)],
                )

            return 0

        lax.fori_loop(0, chunk_trips, wr, 0)

    return pl.kernel(
        body,
        out_type=jax.ShapeDtypeStruct((NC, n, d), jnp.float32),
        mesh=mesh,
        scratch_types=scratch,
    )


def _make_deg(n, e, dw):
    e_per_w = e // NW
    n_iters = e_per_w // CH
    assert e_per_w * NW == e and n_iters * CH == e_per_w
    n_chunks = n // RCH
    assert n_chunks * RCH == n
    chunk_trips = -(-n_chunks // NS)
    kg = dw // LANES

    mesh = plsc.VectorSubcoreMesh(
        core_axis_name="c", subcore_axis_name="s", num_cores=NC, num_subcores=NS
    )
    scratch = [
        pltpu.VMEM((n_iters, CH), jnp.int32),
        pltpu.VMEM((CH, dw), jnp.float32),
        pltpu.VMEM_SHARED((n, dw), jnp.float32),
    ]

    def body(dst_hbm, deg_hbm, dst_v, ones_v, deg_sh):
        c = lax.axis_index("c")
        s = lax.axis_index("s")
        wid = c * NS + s
        zero16 = jnp.zeros((LANES,), jnp.float32)
        ones16 = jnp.ones((LANES,), jnp.float32)

        def fill(val):
            def zo(r, _):
                for k in range(kg):
                    ones_v[r, pl.ds(k * LANES, LANES)] = val
                return 0

            lax.fori_loop(0, CH, zo, 0)

        fill(zero16)

        def zs(ti, _):
            t = s + ti * NS

            @pl.when(t < n_chunks)
            def _():
                pltpu.sync_copy(ones_v, deg_sh.at[pl.ds(t * RCH, RCH)])

            return 0

        lax.fori_loop(0, chunk_trips, zs, 0)
        fill(ones16)

        pltpu.sync_copy(dst_hbm.at[wid], dst_v)
        plsc.subcore_barrier()

        def step(j, _):
            pltpu.sync_copy(ones_v, deg_sh.at[dst_v.at[j]], add=True)
            return 0

        lax.fori_loop(0, n_iters, step, 0)
        plsc.subcore_barrier()

        def wr(ti, _):
            t = s + ti * NS

            @pl.when(t < n_chunks)
            def _():
                pltpu.sync_copy(
                    deg_sh.at[pl.ds(t * RCH, RCH)],
                    deg_hbm.at[c, pl.ds(t * RCH, RCH)],
                )

            return 0

        lax.fori_loop(0, chunk_trips, wr, 0)

    return pl.kernel(
        body,
        out_type=jax.ShapeDtypeStruct((NC, n, dw), jnp.float32),
        mesh=mesh,
        scratch_types=scratch,
    )


def _make_scorer(n, ph, m):
    e_per_w = m // NW
    n_iters = e_per_w // CH
    assert e_per_w * NW == m and n_iters * CH == e_per_w
    kg = ph // LANES

    mesh = plsc.VectorSubcoreMesh(
        core_axis_name="c", subcore_axis_name="s", num_cores=NC, num_subcores=NS
    )
    scratch = [
        pltpu.VMEM((n_iters, CH), jnp.int32),
        pltpu.VMEM((n_iters, CH), jnp.int32),
        pltpu.VMEM((CH, ph), jnp.float32),
        pltpu.VMEM((CH, ph), jnp.float32),
        pltpu.VMEM((ph,), jnp.float32),
        pltpu.VMEM((LANES,), jnp.float32),
        pltpu.VMEM((CH,), jnp.float32),
        pltpu.SemaphoreType.DMA,
        pltpu.SemaphoreType.DMA,
    ]

    def body(sa_hbm, sb_hbm, u_hbm, v_hbm, w2_hbm, b2_hbm, out_hbm,
             u_v, v_v, sa_v, sb_v, w2_v, b2_v, sc_v, sem_a, sem_b):
        c = lax.axis_index("c")
        s = lax.axis_index("s")
        wid = c * NS + s
        pltpu.sync_copy(u_hbm.at[wid], u_v)
        pltpu.sync_copy(v_hbm.at[wid], v_v)
        pltpu.sync_copy(w2_hbm, w2_v)
        pltpu.sync_copy(b2_hbm, b2_v)
        w2r = [w2_v[pl.ds(k * LANES, LANES)] for k in range(kg)]
        b2r = b2_v[...]
        lane = lax.iota(jnp.int32, LANES)
        perms = [lane ^ m for m in (8, 4, 2, 1)]

        dn = lax.GatherDimensionNumbers(
            offset_dims=(), collapsed_slice_dims=(0,), start_index_map=(0,)
        )

        def hsum(vv):
            for p in perms:
                pv = lax.gather(
                    vv, p[:, None], dn, (1,),
                    mode=lax.GatherScatterMode.PROMISE_IN_BOUNDS,
                )
                vv = vv + pv
            return vv

        def step(j, _):
            cpa = pltpu.async_copy(sa_hbm.at[u_v.at[j]], sa_v, sem_a)
            cpb = pltpu.async_copy(sb_hbm.at[v_v.at[j]], sb_v, sem_b)
            cpa.wait()
            cpb.wait()

            def grp(g, _):
                vec = jnp.zeros((LANES,), jnp.float32)
                for i in range(LANES):
                    e = g * LANES + i
                    acc = jnp.zeros((LANES,), jnp.float32)
                    for k in range(kg):
                        a = sa_v[e, pl.ds(k * LANES, LANES)]
                        b = sb_v[e, pl.ds(k * LANES, LANES)]
                        acc = acc + jnp.maximum(a + b, 0.0) * w2r[k]
                    vec = jnp.where(lane == i, hsum(acc), vec)
                sc_v[pl.ds(g * LANES, LANES)] = vec + b2r
                return 0

            lax.fori_loop(0, CH // LANES, grp, 0)
            pltpu.sync_copy(
                sc_v, out_hbm.at[pl.ds(wid * e_per_w + j * CH, CH)]
            )
            return 0

        lax.fori_loop(0, n_iters, step, 0)

    return pl.kernel(
        body,
        out_type=jax.ShapeDtypeStruct((m,), jnp.float32),
        mesh=mesh,
        scratch_types=scratch,
    )


def _tc1_body(x_ref, p_ref, dp_ref, ws_ref, wn_ref, b_ref, h1_ref, deg_ref):
    deg = jnp.maximum(dp_ref[0, :, 0] + dp_ref[1, :, 0], 1.0)[:, None]
    agg = (p_ref[0] + p_ref[1]) / deg
    h = (
        jnp.dot(x_ref[...], ws_ref[...], preferred_element_type=jnp.float32)
        + jnp.dot(agg, wn_ref[...], preferred_element_type=jnp.float32)
        + b_ref[...]
    )
    h1_ref[...] = jnp.maximum(h, 0.0)
    deg_ref[...] = deg


def _tc2_body(x_ref, h1_ref, q_ref, deg_ref, ws_ref, wn_ref, b_ref,
              wp1_ref, bp1_ref, sa_ref, sb_ref):
    d = x_ref.shape[1]
    h = h1_ref.shape[1]
    agg = (q_ref[0] + q_ref[1]) / deg_ref[...]
    h2 = (
        jnp.dot(h1_ref[...], ws_ref[...], preferred_element_type=jnp.float32)
        + jnp.dot(agg, wn_ref[...], preferred_element_type=jnp.float32)
        + b_ref[...]
    )
    wp1 = wp1_ref[...]
    sa_ref[...] = (
        jnp.dot(x_ref[...], wp1[:d], preferred_element_type=jnp.float32)
        + bp1_ref[...]
    )
    sb_ref[...] = jnp.dot(
        h2, wp1[d : d + h], preferred_element_type=jnp.float32
    ) + jnp.dot(x_ref[...], wp1[d + h :], preferred_element_type=jnp.float32)


def kernel(x, edge_index, pos_edge_index, neg_edge_index,
           Wself1, Wneigh1, bs1, Wself2, Wneigh2, bs2,
           Wp1, bp1, Wp2, bp2):
    n, d = x.shape
    h = Wself1.shape[1]
    ph = Wp1.shape[1]
    e = edge_index.shape[1]
    ep = pos_edge_index.shape[1]

    src3d = edge_index[0].reshape(NW, -1, CH)
    dst3d = edge_index[1].reshape(NW, -1, CH)

    seg1 = _make_seg_sum(n, d, e)
    seg2 = _make_seg_sum(n, h, e)

    degp = _make_deg(n, e, d)(dst3d)
    agg1p = seg1(x, src3d, dst3d)

    h1, deg = pl.pallas_call(
        _tc1_body,
        out_shape=(
            jax.ShapeDtypeStruct((n, h), jnp.float32),
            jax.ShapeDtypeStruct((n, 1), jnp.float32),
        ),
    )(x, agg1p, degp, Wself1, Wneigh1, bs1.reshape(1, h))

    agg2p = seg2(h1, src3d, dst3d)

    sa, sb = pl.pallas_call(
        _tc2_body,
        out_shape=(
            jax.ShapeDtypeStruct((n, ph), jnp.float32),
            jax.ShapeDtypeStruct((n, ph), jnp.float32),
        ),
    )(x, h1, agg2p, deg, Wself2, Wneigh2, bs2.reshape(1, h),
      Wp1, bp1.reshape(1, ph))

    u3d = jnp.concatenate(
        [pos_edge_index[0], neg_edge_index[0]]
    ).reshape(NW, -1, CH)
    v3d = jnp.concatenate(
        [pos_edge_index[1], neg_edge_index[1]]
    ).reshape(NW, -1, CH)
    w2 = Wp2[:, 0]
    b2v = jnp.full((LANES,), bp2[0], jnp.float32)

    scorer = _make_scorer(n, ph, 2 * ep)
    scores = scorer(sa, sb, u3d, v3d, w2, b2v)
    return scores[:ep], scores[ep:]


# speedup vs baseline: 4.4633x; 1.4227x over previous
import jax
import jax.numpy as jnp
from jax import lax
from jax.experimental import pallas as pl
from jax.experimental.pallas import tpu as pltpu
from jax.experimental.pallas import tpu_sc as plsc

NC = 2
NS = 16
LANES = 16
NW = NC * NS
CHS = 100
WR = 80
CH = 80


def _mesh():
    return plsc.VectorSubcoreMesh(
        core_axis_name="c", subcore_axis_name="s", num_cores=NC, num_subcores=NS
    )


def _make_seg_sum(n, d, e):
    e_per_w = e // NW
    n_iters = e_per_w // CHS
    halfs = n_iters // 2
    pairs = n_iters // 2
    assert e_per_w * NW == e and pairs * 2 * CHS == e_per_w
    n_chunks = n // WR
    assert n_chunks * WR == n
    chunk_trips = -(-n_chunks // NS)
    kg = d // LANES

    scratch = [
        pltpu.VMEM((halfs, CHS), jnp.int32),
        pltpu.VMEM((n_iters, CHS), jnp.int32),
        pltpu.VMEM((CHS, d), jnp.float32),
        pltpu.VMEM((CHS, d), jnp.float32),
        pltpu.VMEM_SHARED((n, d), jnp.float32),
        pltpu.SemaphoreType.DMA,
        pltpu.SemaphoreType.DMA,
    ]

    def body(x_hbm, src_hbm, dst_hbm, out_hbm,
             src_v, dst_v, rows0, rows1, agg_sh, sem0, sem1):
        c = lax.axis_index("c")
        s = lax.axis_index("s")
        wid = c * NS + s
        zero16 = jnp.zeros((LANES,), jnp.float32)
        rows = (rows0, rows1)
        sems = (sem0, sem1)

        def zb(r, _):
            for k in range(kg):
                rows0[r, pl.ds(k * LANES, LANES)] = zero16
            return 0

        lax.fori_loop(0, WR, zb, 0)

        def zs(ti, _):
            t = s + ti * NS

            @pl.when(t < n_chunks)
            def _():
                pltpu.sync_copy(
                    rows0.at[pl.ds(0, WR)], agg_sh.at[pl.ds(t * WR, WR)]
                )

            return 0

        lax.fori_loop(0, chunk_trips, zs, 0)

        pltpu.sync_copy(src_hbm.at[wid, 0], src_v)
        pltpu.sync_copy(dst_hbm.at[wid], dst_v)
        plsc.subcore_barrier()

        pltpu.async_copy(x_hbm.at[src_v.at[0]], rows0, sem0)

        def half(j, b):
            @pl.when(j == halfs - 1)
            def _():
                pltpu.sync_copy(src_hbm.at[wid, 1], src_v)

            @pl.when(j + 1 < n_iters)
            def _():
                row = lax.rem(j + 1, halfs)
                pltpu.async_copy(
                    x_hbm.at[src_v.at[row]], rows[1 - b], sems[1 - b]
                )

            pltpu.make_async_copy(
                x_hbm.at[src_v.at[lax.rem(j, halfs)]], rows[b], sems[b]
            ).wait()
            pltpu.sync_copy(rows[b], agg_sh.at[dst_v.at[j]], add=True)

        def pair(jj, _):
            half(2 * jj, 0)
            half(2 * jj + 1, 1)
            return 0

        lax.fori_loop(0, pairs, pair, 0)
        plsc.subcore_barrier()

        def wr(ti, _):
            t = s + ti * NS

            @pl.when(t < n_chunks)
            def _():
                pltpu.sync_copy(
                    agg_sh.at[pl.ds(t * WR, WR)],
                    out_hbm.at[c, pl.ds(t * WR, WR)],
                )

            return 0

        lax.fori_loop(0, chunk_trips, wr, 0)

    return pl.kernel(
        body,
        out_type=jax.ShapeDtypeStruct((NC, n, d), jnp.float32),
        mesh=_mesh(),
        scratch_types=scratch,
    )


def _make_deg(n, e, dw):
    e_per_w = e // NW
    n_iters = e_per_w // CHS
    grp = 10
    n_grps = n_iters // grp
    assert e_per_w * NW == e and n_grps * grp * CHS == e_per_w
    n_chunks = n // WR
    assert n_chunks * WR == n
    chunk_trips = -(-n_chunks // NS)
    kg = dw // LANES

    scratch = [
        pltpu.VMEM((n_iters, CHS), jnp.int32),
        pltpu.VMEM((CHS, dw), jnp.float32),
        pltpu.VMEM_SHARED((n, dw), jnp.float32),
        pltpu.SemaphoreType.DMA,
    ]

    def body(dst_hbm, deg_hbm, dst_v, ones_v, deg_sh, sem):
        c = lax.axis_index("c")
        s = lax.axis_index("s")
        wid = c * NS + s
        zero16 = jnp.zeros((LANES,), jnp.float32)
        ones16 = jnp.ones((LANES,), jnp.float32)

        def fill(val):
            def zo(r, _):
                for k in range(kg):
                    ones_v[r, pl.ds(k * LANES, LANES)] = val
                return 0

            lax.fori_loop(0, CHS, zo, 0)

        fill(zero16)

        def zs(ti, _):
            t = s + ti * NS

            @pl.when(t < n_chunks)
            def _():
                pltpu.sync_copy(
                    ones_v.at[pl.ds(0, WR)], deg_sh.at[pl.ds(t * WR, WR)]
                )

            return 0

        lax.fori_loop(0, chunk_trips, zs, 0)
        fill(ones16)

        pltpu.sync_copy(dst_hbm.at[wid], dst_v)
        plsc.subcore_barrier()

        def step(g, _):
            for k in range(grp):
                pltpu.async_copy(
                    ones_v, deg_sh.at[dst_v.at[g * grp + k]], sem, add=True
                )
            for k in range(grp):
                pltpu.make_async_copy(
                    ones_v, deg_sh.at[dst_v.at[g * grp + k]], sem
                ).wait()
            return 0

        lax.fori_loop(0, n_grps, step, 0)
        plsc.subcore_barrier()

        def wr(ti, _):
            t = s + ti * NS

            @pl.when(t < n_chunks)
            def _():
                pltpu.sync_copy(
                    deg_sh.at[pl.ds(t * WR, WR)],
                    deg_hbm.at[c, pl.ds(t * WR, WR)],
                )

            return 0

        lax.fori_loop(0, chunk_trips, wr, 0)

    return pl.kernel(
        body,
        out_type=jax.ShapeDtypeStruct((NC, n, dw), jnp.float32),
        mesh=_mesh(),
        scratch_types=scratch,
    )


def _make_scorer(n, ph, m):
    e_per_w = m // NW
    n_iters = e_per_w // CH
    half_rows = n_iters // 2
    pairs = n_iters // 2
    assert e_per_w * NW == m and pairs * 2 * CH == e_per_w
    kg = ph // LANES

    scratch = [
        pltpu.VMEM((half_rows, CH), jnp.int32),
        pltpu.VMEM((half_rows, CH), jnp.int32),
        pltpu.VMEM((CH, ph), jnp.float32),
        pltpu.VMEM((CH, ph), jnp.float32),
        pltpu.VMEM((CH, ph), jnp.float32),
        pltpu.VMEM((CH, ph), jnp.float32),
        pltpu.VMEM((ph,), jnp.float32),
        pltpu.VMEM((LANES,), jnp.float32),
        pltpu.VMEM((CH,), jnp.float32),
        pltpu.SemaphoreType.DMA,
        pltpu.SemaphoreType.DMA,
        pltpu.SemaphoreType.DMA,
        pltpu.SemaphoreType.DMA,
    ]

    def body(sa_hbm, sb_hbm, u_hbm, v_hbm, w2_hbm, b2_hbm, out_hbm,
             u_v, v_v, sa0, sa1, sb0, sb1, w2_v, b2_v, sc_v,
             sa_sem0, sa_sem1, sb_sem0, sb_sem1):
        c = lax.axis_index("c")
        s = lax.axis_index("s")
        wid = c * NS + s
        sa_bufs = (sa0, sa1)
        sb_bufs = (sb0, sb1)
        sa_sems = (sa_sem0, sa_sem1)
        sb_sems = (sb_sem0, sb_sem1)

        pltpu.sync_copy(u_hbm.at[wid, 0], u_v)
        pltpu.sync_copy(v_hbm.at[wid, 0], v_v)
        pltpu.sync_copy(w2_hbm, w2_v)
        pltpu.sync_copy(b2_hbm, b2_v)
        w2r = [w2_v[pl.ds(k * LANES, LANES)] for k in range(kg)]
        b2r = b2_v[...]
        lane = lax.iota(jnp.int32, LANES)
        perms = [lane ^ mm for mm in (8, 4, 2, 1)]
        dn = lax.GatherDimensionNumbers(
            offset_dims=(), collapsed_slice_dims=(0,), start_index_map=(0,)
        )

        def hsum(vv):
            for p in perms:
                pv = lax.gather(
                    vv, p[:, None], dn, (1,),
                    mode=lax.GatherScatterMode.PROMISE_IN_BOUNDS,
                )
                vv = vv + pv
            return vv

        pltpu.async_copy(sa_hbm.at[u_v.at[0]], sa0, sa_sem0)
        pltpu.async_copy(sb_hbm.at[v_v.at[0]], sb0, sb_sem0)

        def half(j, b):
            @pl.when(j == half_rows - 1)
            def _():
                pltpu.sync_copy(u_hbm.at[wid, 1], u_v)
                pltpu.sync_copy(v_hbm.at[wid, 1], v_v)

            @pl.when(j + 1 < n_iters)
            def _():
                row = lax.rem(j + 1, half_rows)
                pltpu.async_copy(
                    sa_hbm.at[u_v.at[row]], sa_bufs[1 - b], sa_sems[1 - b]
                )
                pltpu.async_copy(
                    sb_hbm.at[v_v.at[row]], sb_bufs[1 - b], sb_sems[1 - b]
                )

            pltpu.make_async_copy(
                sa_hbm.at[pl.ds(0, CH)], sa_bufs[b], sa_sems[b]
            ).wait()
            pltpu.make_async_copy(
                sb_hbm.at[pl.ds(0, CH)], sb_bufs[b], sb_sems[b]
            ).wait()
            sa_v = sa_bufs[b]
            sb_v = sb_bufs[b]

            def grp(g, _):
                vec = jnp.zeros((LANES,), jnp.float32)
                for i in range(LANES):
                    ee = g * LANES + i
                    acc = jnp.zeros((LANES,), jnp.float32)
                    for k in range(kg):
                        a = sa_v[ee, pl.ds(k * LANES, LANES)]
                        bb = sb_v[ee, pl.ds(k * LANES, LANES)]
                        acc = acc + jnp.maximum(a + bb, 0.0) * w2r[k]
                    vec = jnp.where(lane == i, hsum(acc), vec)
                sc_v[pl.ds(g * LANES, LANES)] = vec + b2r
                return 0

            lax.fori_loop(0, CH // LANES, grp, 0)
            pltpu.sync_copy(
                sc_v, out_hbm.at[pl.ds(wid * e_per_w + j * CH, CH)]
            )

        def pair(jj, _):
            half(2 * jj, 0)
            half(2 * jj + 1, 1)
            return 0

        lax.fori_loop(0, pairs, pair, 0)

    return pl.kernel(
        body,
        out_type=jax.ShapeDtypeStruct((m,), jnp.float32),
        mesh=_mesh(),
        scratch_types=scratch,
    )


def _tc1_body(x_ref, p_ref, dp_ref, ws_ref, wn_ref, b_ref, h1_ref, deg_ref):
    deg = jnp.maximum(dp_ref[0, :, 0] + dp_ref[1, :, 0], 1.0)[:, None]
    agg = (p_ref[0] + p_ref[1]) / deg
    h = (
        jnp.dot(x_ref[...], ws_ref[...], preferred_element_type=jnp.float32)
        + jnp.dot(agg, wn_ref[...], preferred_element_type=jnp.float32)
        + b_ref[...]
    )
    h1_ref[...] = jnp.maximum(h, 0.0)
    deg_ref[...] = deg


def _tc2_body(x_ref, h1_ref, q_ref, deg_ref, ws_ref, wn_ref, b_ref,
              wp1_ref, bp1_ref, sa_ref, sb_ref):
    d = x_ref.shape[1]
    h = h1_ref.shape[1]
    agg = (q_ref[0] + q_ref[1]) / deg_ref[...]
    h2 = (
        jnp.dot(h1_ref[...], ws_ref[...], preferred_element_type=jnp.float32)
        + jnp.dot(agg, wn_ref[...], preferred_element_type=jnp.float32)
        + b_ref[...]
    )
    wp1 = wp1_ref[...]
    sa_ref[...] = (
        jnp.dot(x_ref[...], wp1[:d], preferred_element_type=jnp.float32)
        + bp1_ref[...]
    )
    sb_ref[...] = jnp.dot(
        h2, wp1[d : d + h], preferred_element_type=jnp.float32
    ) + jnp.dot(x_ref[...], wp1[d + h :], preferred_element_type=jnp.float32)


def kernel(x, edge_index, pos_edge_index, neg_edge_index,
           Wself1, Wneigh1, bs1, Wself2, Wneigh2, bs2,
           Wp1, bp1, Wp2, bp2):
    n, d = x.shape
    h = Wself1.shape[1]
    ph = Wp1.shape[1]
    e = edge_index.shape[1]
    ep = pos_edge_index.shape[1]

    src4d = edge_index[0].reshape(NW, 2, -1, CHS)
    dst3d = edge_index[1].reshape(NW, -1, CHS)

    seg1 = _make_seg_sum(n, d, e)
    seg2 = _make_seg_sum(n, h, e)

    degp = _make_deg(n, e, d)(dst3d)
    agg1p = seg1(x, src4d, dst3d)

    h1, deg = pl.pallas_call(
        _tc1_body,
        out_shape=(
            jax.ShapeDtypeStruct((n, h), jnp.float32),
            jax.ShapeDtypeStruct((n, 1), jnp.float32),
        ),
    )(x, agg1p, degp, Wself1, Wneigh1, bs1.reshape(1, h))

    agg2p = seg2(h1, src4d, dst3d)

    sa, sb = pl.pallas_call(
        _tc2_body,
        out_shape=(
            jax.ShapeDtypeStruct((n, ph), jnp.float32),
            jax.ShapeDtypeStruct((n, ph), jnp.float32),
        ),
    )(x, h1, agg2p, deg, Wself2, Wneigh2, bs2.reshape(1, h),
      Wp1, bp1.reshape(1, ph))

    u4d = jnp.concatenate(
        [pos_edge_index[0], neg_edge_index[0]]
    ).reshape(NW, 2, -1, CH)
    v4d = jnp.concatenate(
        [pos_edge_index[1], neg_edge_index[1]]
    ).reshape(NW, 2, -1, CH)
    w2 = Wp2[:, 0]
    b2v = jnp.full((LANES,), bp2[0], jnp.float32)

    scorer = _make_scorer(n, ph, 2 * ep)
    scores = scorer(sa, sb, u4d, v4d, w2, b2v)
    return scores[:ep], scores[ep:]


# speedup vs baseline: 5.1759x; 1.1597x over previous
import jax
import jax.numpy as jnp
from jax import lax
from jax.experimental import pallas as pl
from jax.experimental.pallas import tpu as pltpu
from jax.experimental.pallas import tpu_sc as plsc

NC = 2
NS = 16
LANES = 16
NW = NC * NS
CHS = 100
WR = 80
CH = 80


def _mesh():
    return plsc.VectorSubcoreMesh(
        core_axis_name="c", subcore_axis_name="s", num_cores=NC, num_subcores=NS
    )


def _make_seg_sum(n, d, e):
    e_per_w = e // NW
    n_iters = e_per_w // CHS
    halfs = n_iters // 2
    pairs = n_iters // 2
    assert e_per_w * NW == e and pairs * 2 * CHS == e_per_w
    n_chunks = n // WR
    assert n_chunks * WR == n
    chunk_trips = -(-n_chunks // NS)
    kg = d // LANES

    scratch = [
        pltpu.VMEM((halfs, CHS), jnp.int32),
        pltpu.VMEM((n_iters, CHS), jnp.int32),
        pltpu.VMEM((CHS, d), jnp.float32),
        pltpu.VMEM((CHS, d), jnp.float32),
        pltpu.VMEM_SHARED((n, d), jnp.float32),
        pltpu.SemaphoreType.DMA,
        pltpu.SemaphoreType.DMA,
    ]

    def body(x_hbm, src_hbm, dst_hbm, out_hbm,
             src_v, dst_v, rows0, rows1, agg_sh, sem0, sem1):
        c = lax.axis_index("c")
        s = lax.axis_index("s")
        wid = c * NS + s
        zero16 = jnp.zeros((LANES,), jnp.float32)
        rows = (rows0, rows1)
        sems = (sem0, sem1)

        def zb(r, _):
            for k in range(kg):
                rows0[r, pl.ds(k * LANES, LANES)] = zero16
            return 0

        lax.fori_loop(0, WR, zb, 0)

        def zs(ti, _):
            t = s + ti * NS

            @pl.when(t < n_chunks)
            def _():
                pltpu.sync_copy(
                    rows0.at[pl.ds(0, WR)], agg_sh.at[pl.ds(t * WR, WR)]
                )

            return 0

        lax.fori_loop(0, chunk_trips, zs, 0)

        pltpu.sync_copy(src_hbm.at[wid, 0], src_v)
        pltpu.sync_copy(dst_hbm.at[wid], dst_v)
        plsc.subcore_barrier()

        pltpu.async_copy(x_hbm.at[src_v.at[0]], rows0, sem0)

        def half(j, b):
            @pl.when(j == halfs - 1)
            def _():
                pltpu.sync_copy(src_hbm.at[wid, 1], src_v)

            @pl.when(j + 1 < n_iters)
            def _():
                row = lax.rem(j + 1, halfs)
                pltpu.async_copy(
                    x_hbm.at[src_v.at[row]], rows[1 - b], sems[1 - b]
                )

            pltpu.make_async_copy(
                x_hbm.at[src_v.at[lax.rem(j, halfs)]], rows[b], sems[b]
            ).wait()
            pltpu.sync_copy(rows[b], agg_sh.at[dst_v.at[j]], add=True)

        def pair(jj, _):
            half(2 * jj, 0)
            half(2 * jj + 1, 1)
            return 0

        lax.fori_loop(0, pairs, pair, 0)
        plsc.subcore_barrier()

        def wr(ti, _):
            t = s + ti * NS

            @pl.when(t < n_chunks)
            def _():
                pltpu.sync_copy(
                    agg_sh.at[pl.ds(t * WR, WR)],
                    out_hbm.at[c, pl.ds(t * WR, WR)],
                )

            return 0

        lax.fori_loop(0, chunk_trips, wr, 0)

    return pl.kernel(
        body,
        out_type=jax.ShapeDtypeStruct((NC, n, d), jnp.float32),
        mesh=_mesh(),
        scratch_types=scratch,
    )


def _make_deg(n, e, dw):
    e_per_w = e // NW
    n_iters = e_per_w // CHS
    grp = 10
    n_grps = n_iters // grp
    assert e_per_w * NW == e and n_grps * grp * CHS == e_per_w
    n_chunks = n // WR
    assert n_chunks * WR == n
    chunk_trips = -(-n_chunks // NS)
    kg = dw // LANES

    scratch = [
        pltpu.VMEM((n_iters, CHS), jnp.int32),
        pltpu.VMEM((CHS, dw), jnp.float32),
        pltpu.VMEM_SHARED((n, dw), jnp.float32),
        pltpu.SemaphoreType.DMA,
    ]

    def body(dst_hbm, deg_hbm, dst_v, ones_v, deg_sh, sem):
        c = lax.axis_index("c")
        s = lax.axis_index("s")
        wid = c * NS + s
        zero16 = jnp.zeros((LANES,), jnp.float32)
        ones16 = jnp.ones((LANES,), jnp.float32)

        def fill(val):
            def zo(r, _):
                for k in range(kg):
                    ones_v[r, pl.ds(k * LANES, LANES)] = val
                return 0

            lax.fori_loop(0, CHS, zo, 0)

        fill(zero16)

        def zs(ti, _):
            t = s + ti * NS

            @pl.when(t < n_chunks)
            def _():
                pltpu.sync_copy(
                    ones_v.at[pl.ds(0, WR)], deg_sh.at[pl.ds(t * WR, WR)]
                )

            return 0

        lax.fori_loop(0, chunk_trips, zs, 0)
        fill(ones16)

        pltpu.sync_copy(dst_hbm.at[wid], dst_v)
        plsc.subcore_barrier()

        def step(g, _):
            for k in range(grp):
                pltpu.async_copy(
                    ones_v, deg_sh.at[dst_v.at[g * grp + k]], sem, add=True
                )
            for k in range(grp):
                pltpu.make_async_copy(
                    ones_v, deg_sh.at[dst_v.at[g * grp + k]], sem
                ).wait()
            return 0

        lax.fori_loop(0, n_grps, step, 0)
        plsc.subcore_barrier()

        def wr(ti, _):
            t = s + ti * NS

            @pl.when(t < n_chunks)
            def _():
                pltpu.sync_copy(
                    deg_sh.at[pl.ds(t * WR, WR)],
                    deg_hbm.at[c, pl.ds(t * WR, WR)],
                )

            return 0

        lax.fori_loop(0, chunk_trips, wr, 0)

    return pl.kernel(
        body,
        out_type=jax.ShapeDtypeStruct((NC, n, dw), jnp.float32),
        mesh=_mesh(),
        scratch_types=scratch,
    )


def _make_scorer(n, ph, m):
    e_per_w = m // NW
    n_iters = e_per_w // CH
    half_rows = n_iters // 2
    pairs = n_iters // 2
    assert e_per_w * NW == m and pairs * 2 * CH == e_per_w
    kg = ph // LANES

    scratch = [
        pltpu.VMEM((half_rows, CH), jnp.int32),
        pltpu.VMEM((half_rows, CH), jnp.int32),
        pltpu.VMEM((CH, ph), jnp.float32),
        pltpu.VMEM((CH, ph), jnp.float32),
        pltpu.VMEM((CH, ph), jnp.float32),
        pltpu.VMEM((CH, ph), jnp.float32),
        pltpu.VMEM((ph,), jnp.float32),
        pltpu.VMEM((LANES,), jnp.float32),
        pltpu.VMEM((CH,), jnp.float32),
        pltpu.SemaphoreType.DMA,
        pltpu.SemaphoreType.DMA,
        pltpu.SemaphoreType.DMA,
        pltpu.SemaphoreType.DMA,
    ]

    def body(sa_hbm, sb_hbm, u_hbm, v_hbm, w2_hbm, b2_hbm, out_hbm,
             u_v, v_v, sa0, sa1, sb0, sb1, w2_v, b2_v, sc_v,
             sa_sem0, sa_sem1, sb_sem0, sb_sem1):
        c = lax.axis_index("c")
        s = lax.axis_index("s")
        wid = c * NS + s
        sa_bufs = (sa0, sa1)
        sb_bufs = (sb0, sb1)
        sa_sems = (sa_sem0, sa_sem1)
        sb_sems = (sb_sem0, sb_sem1)

        pltpu.sync_copy(u_hbm.at[wid, 0], u_v)
        pltpu.sync_copy(v_hbm.at[wid, 0], v_v)
        pltpu.sync_copy(w2_hbm, w2_v)
        pltpu.sync_copy(b2_hbm, b2_v)
        w2r = [w2_v[pl.ds(k * LANES, LANES)] for k in range(kg)]
        b2r = b2_v[...]
        lane = lax.iota(jnp.int32, LANES)
        pidx = {mm: (lane ^ mm)[:, None] for mm in (8, 4, 2, 1)}
        m1 = (lane & 1) != 0
        m2 = (lane & 2) != 0
        bsel = [(lane >> 2) == b4 for b4 in range(4)]
        dn = lax.GatherDimensionNumbers(
            offset_dims=(), collapsed_slice_dims=(0,), start_index_map=(0,)
        )

        def perm(vv, mm):
            return lax.gather(
                vv, pidx[mm], dn, (1,),
                mode=lax.GatherScatterMode.PROMISE_IN_BOUNDS,
            )

        pltpu.async_copy(sa_hbm.at[u_v.at[0]], sa0, sa_sem0)
        pltpu.async_copy(sb_hbm.at[v_v.at[0]], sb0, sb_sem0)

        def half(j, b):
            @pl.when(j == half_rows - 1)
            def _():
                pltpu.sync_copy(u_hbm.at[wid, 1], u_v)
                pltpu.sync_copy(v_hbm.at[wid, 1], v_v)

            @pl.when(j + 1 < n_iters)
            def _():
                row = lax.rem(j + 1, half_rows)
                pltpu.async_copy(
                    sa_hbm.at[u_v.at[row]], sa_bufs[1 - b], sa_sems[1 - b]
                )
                pltpu.async_copy(
                    sb_hbm.at[v_v.at[row]], sb_bufs[1 - b], sb_sems[1 - b]
                )

            pltpu.make_async_copy(
                sa_hbm.at[pl.ds(0, CH)], sa_bufs[b], sa_sems[b]
            ).wait()
            pltpu.make_async_copy(
                sb_hbm.at[pl.ds(0, CH)], sb_bufs[b], sb_sems[b]
            ).wait()
            sa_v = sa_bufs[b]
            sb_v = sb_bufs[b]

            def grp(g, _):
                accs = [jnp.zeros((LANES,), jnp.float32)] * LANES
                for k in range(kg):
                    w = w2r[k]
                    for i in range(LANES):
                        ee = g * LANES + i
                        a = sa_v[ee, pl.ds(k * LANES, LANES)]
                        bb = sb_v[ee, pl.ds(k * LANES, LANES)]
                        accs[i] = accs[i] + jnp.maximum(a + bb, 0.0) * w
                res = jnp.zeros((LANES,), jnp.float32)
                for b4 in range(4):
                    v0, v1, v2, v3 = accs[4 * b4 : 4 * b4 + 4]
                    c01 = jnp.where(m1, v1, v0) + perm(jnp.where(m1, v0, v1), 1)
                    c23 = jnp.where(m1, v3, v2) + perm(jnp.where(m1, v2, v3), 1)
                    cc = jnp.where(m2, c23, c01) + perm(jnp.where(m2, c01, c23), 2)
                    d = cc + perm(cc, 4)
                    e = d + perm(d, 8)
                    res = jnp.where(bsel[b4], e, res)
                sc_v[pl.ds(g * LANES, LANES)] = res + b2r
                return 0

            lax.fori_loop(0, CH // LANES, grp, 0)
            pltpu.sync_copy(
                sc_v, out_hbm.at[pl.ds(wid * e_per_w + j * CH, CH)]
            )

        def pair(jj, _):
            half(2 * jj, 0)
            half(2 * jj + 1, 1)
            return 0

        lax.fori_loop(0, pairs, pair, 0)

    return pl.kernel(
        body,
        out_type=jax.ShapeDtypeStruct((m,), jnp.float32),
        mesh=_mesh(),
        scratch_types=scratch,
    )


def _tc1_body(x_ref, p_ref, dp_ref, ws_ref, wn_ref, b_ref, h1_ref, deg_ref):
    deg = jnp.maximum(dp_ref[0, :, 0] + dp_ref[1, :, 0], 1.0)[:, None]
    agg = (p_ref[0] + p_ref[1]) / deg
    h = (
        jnp.dot(x_ref[...], ws_ref[...], preferred_element_type=jnp.float32)
        + jnp.dot(agg, wn_ref[...], preferred_element_type=jnp.float32)
        + b_ref[...]
    )
    h1_ref[...] = jnp.maximum(h, 0.0)
    deg_ref[...] = deg


def _tc2_body(x_ref, h1_ref, q_ref, deg_ref, ws_ref, wn_ref, b_ref,
              wp1_ref, bp1_ref, sa_ref, sb_ref):
    d = x_ref.shape[1]
    h = h1_ref.shape[1]
    agg = (q_ref[0] + q_ref[1]) / deg_ref[...]
    h2 = (
        jnp.dot(h1_ref[...], ws_ref[...], preferred_element_type=jnp.float32)
        + jnp.dot(agg, wn_ref[...], preferred_element_type=jnp.float32)
        + b_ref[...]
    )
    wp1 = wp1_ref[...]
    sa_ref[...] = (
        jnp.dot(x_ref[...], wp1[:d], preferred_element_type=jnp.float32)
        + bp1_ref[...]
    )
    sb_ref[...] = jnp.dot(
        h2, wp1[d : d + h], preferred_element_type=jnp.float32
    ) + jnp.dot(x_ref[...], wp1[d + h :], preferred_element_type=jnp.float32)


def kernel(x, edge_index, pos_edge_index, neg_edge_index,
           Wself1, Wneigh1, bs1, Wself2, Wneigh2, bs2,
           Wp1, bp1, Wp2, bp2):
    n, d = x.shape
    h = Wself1.shape[1]
    ph = Wp1.shape[1]
    e = edge_index.shape[1]
    ep = pos_edge_index.shape[1]

    src4d = edge_index[0].reshape(NW, 2, -1, CHS)
    dst3d = edge_index[1].reshape(NW, -1, CHS)

    seg1 = _make_seg_sum(n, d, e)
    seg2 = _make_seg_sum(n, h, e)

    degp = _make_deg(n, e, d)(dst3d)
    agg1p = seg1(x, src4d, dst3d)

    h1, deg = pl.pallas_call(
        _tc1_body,
        out_shape=(
            jax.ShapeDtypeStruct((n, h), jnp.float32),
            jax.ShapeDtypeStruct((n, 1), jnp.float32),
        ),
    )(x, agg1p, degp, Wself1, Wneigh1, bs1.reshape(1, h))

    agg2p = seg2(h1, src4d, dst3d)

    sa, sb = pl.pallas_call(
        _tc2_body,
        out_shape=(
            jax.ShapeDtypeStruct((n, ph), jnp.float32),
            jax.ShapeDtypeStruct((n, ph), jnp.float32),
        ),
    )(x, h1, agg2p, deg, Wself2, Wneigh2, bs2.reshape(1, h),
      Wp1, bp1.reshape(1, ph))

    u4d = jnp.concatenate(
        [pos_edge_index[0], neg_edge_index[0]]
    ).reshape(NW, 2, -1, CH)
    v4d = jnp.concatenate(
        [pos_edge_index[1], neg_edge_index[1]]
    ).reshape(NW, 2, -1, CH)
    w2 = Wp2[:, 0]
    b2v = jnp.full((LANES,), bp2[0], jnp.float32)

    scorer = _make_scorer(n, ph, 2 * ep)
    scores = scorer(sa, sb, u4d, v4d, w2, b2v)
    return scores[:ep], scores[ep:]


# speedup vs baseline: 6.1214x; 1.1827x over previous
import jax
import jax.numpy as jnp
from jax import lax
from jax.experimental import pallas as pl
from jax.experimental.pallas import tpu as pltpu
from jax.experimental.pallas import tpu_sc as plsc

NC = 2
NS = 16
LANES = 16
NW = NC * NS
CHS = 100
WR = 80
CH = 80


def _mesh():
    return plsc.VectorSubcoreMesh(
        core_axis_name="c", subcore_axis_name="s", num_cores=NC, num_subcores=NS
    )


def _make_seg_sum(n, d, e):
    e_per_w = e // NW
    n_iters = e_per_w // CHS
    halfs = n_iters // 2
    pairs = n_iters // 2
    assert e_per_w * NW == e and pairs * 2 * CHS == e_per_w
    n_chunks = n // WR
    assert n_chunks * WR == n
    chunk_trips = -(-n_chunks // NS)
    kg = d // LANES

    scratch = [
        pltpu.VMEM((halfs, CHS), jnp.int32),
        pltpu.VMEM((n_iters, CHS), jnp.int32),
        pltpu.VMEM((CHS, d), jnp.float32),
        pltpu.VMEM((CHS, d), jnp.float32),
        pltpu.VMEM_SHARED((n, d), jnp.float32),
        pltpu.SemaphoreType.DMA,
        pltpu.SemaphoreType.DMA,
    ]

    def body(x_hbm, src_hbm, dst_hbm, out_hbm,
             src_v, dst_v, rows0, rows1, agg_sh, sem0, sem1):
        c = lax.axis_index("c")
        s = lax.axis_index("s")
        wid = c * NS + s
        zero16 = jnp.zeros((LANES,), jnp.float32)
        rows = (rows0, rows1)
        sems = (sem0, sem1)

        def zb(r, _):
            for k in range(kg):
                rows0[r, pl.ds(k * LANES, LANES)] = zero16
            return 0

        lax.fori_loop(0, WR, zb, 0)

        def zs(ti, _):
            t = s + ti * NS

            @pl.when(t < n_chunks)
            def _():
                pltpu.sync_copy(
                    rows0.at[pl.ds(0, WR)], agg_sh.at[pl.ds(t * WR, WR)]
                )

            return 0

        lax.fori_loop(0, chunk_trips, zs, 0)

        pltpu.sync_copy(src_hbm.at[wid, 0], src_v)
        pltpu.sync_copy(dst_hbm.at[wid], dst_v)
        plsc.subcore_barrier()

        pltpu.async_copy(x_hbm.at[src_v.at[0]], rows0, sem0)

        def half(j, b):
            @pl.when(j == halfs - 1)
            def _():
                pltpu.sync_copy(src_hbm.at[wid, 1], src_v)

            @pl.when(j + 1 < n_iters)
            def _():
                row = lax.rem(j + 1, halfs)
                pltpu.async_copy(
                    x_hbm.at[src_v.at[row]], rows[1 - b], sems[1 - b]
                )

            pltpu.make_async_copy(
                x_hbm.at[src_v.at[lax.rem(j, halfs)]], rows[b], sems[b]
            ).wait()
            pltpu.sync_copy(rows[b], agg_sh.at[dst_v.at[j]], add=True)

        def pair(jj, _):
            half(2 * jj, 0)
            half(2 * jj + 1, 1)
            return 0

        lax.fori_loop(0, pairs, pair, 0)
        plsc.subcore_barrier()

        def wr(ti, _):
            t = s + ti * NS

            @pl.when(t < n_chunks)
            def _():
                pltpu.sync_copy(
                    agg_sh.at[pl.ds(t * WR, WR)],
                    out_hbm.at[c, pl.ds(t * WR, WR)],
                )

            return 0

        lax.fori_loop(0, chunk_trips, wr, 0)

    return pl.kernel(
        body,
        out_type=jax.ShapeDtypeStruct((NC, n, d), jnp.float32),
        mesh=_mesh(),
        scratch_types=scratch,
    )


def _make_deg(n, e, dw):
    e_per_w = e // NW
    n_iters = e_per_w // CHS
    grp = 10
    n_grps = n_iters // grp
    assert e_per_w * NW == e and n_grps * grp * CHS == e_per_w
    n_chunks = n // WR
    assert n_chunks * WR == n
    chunk_trips = -(-n_chunks // NS)
    kg = dw // LANES

    scratch = [
        pltpu.VMEM((n_iters, CHS), jnp.int32),
        pltpu.VMEM((CHS, dw), jnp.float32),
        pltpu.VMEM_SHARED((n, dw), jnp.float32),
        pltpu.SemaphoreType.DMA,
    ]

    def body(dst_hbm, deg_hbm, dst_v, ones_v, deg_sh, sem):
        c = lax.axis_index("c")
        s = lax.axis_index("s")
        wid = c * NS + s
        zero16 = jnp.zeros((LANES,), jnp.float32)
        ones16 = jnp.ones((LANES,), jnp.float32)

        def fill(val):
            def zo(r, _):
                for k in range(kg):
                    ones_v[r, pl.ds(k * LANES, LANES)] = val
                return 0

            lax.fori_loop(0, CHS, zo, 0)

        fill(zero16)

        def zs(ti, _):
            t = s + ti * NS

            @pl.when(t < n_chunks)
            def _():
                pltpu.sync_copy(
                    ones_v.at[pl.ds(0, WR)], deg_sh.at[pl.ds(t * WR, WR)]
                )

            return 0

        lax.fori_loop(0, chunk_trips, zs, 0)
        fill(ones16)

        pltpu.sync_copy(dst_hbm.at[wid], dst_v)
        plsc.subcore_barrier()

        def step(g, _):
            for k in range(grp):
                pltpu.async_copy(
                    ones_v, deg_sh.at[dst_v.at[g * grp + k]], sem, add=True
                )
            for k in range(grp):
                pltpu.make_async_copy(
                    ones_v, deg_sh.at[dst_v.at[g * grp + k]], sem
                ).wait()
            return 0

        lax.fori_loop(0, n_grps, step, 0)
        plsc.subcore_barrier()

        def wr(ti, _):
            t = s + ti * NS

            @pl.when(t < n_chunks)
            def _():
                pltpu.sync_copy(
                    deg_sh.at[pl.ds(t * WR, WR)],
                    deg_hbm.at[c, pl.ds(t * WR, WR)],
                )

            return 0

        lax.fori_loop(0, chunk_trips, wr, 0)

    return pl.kernel(
        body,
        out_type=jax.ShapeDtypeStruct((NC, n, dw), jnp.float32),
        mesh=_mesh(),
        scratch_types=scratch,
    )


def _make_scorer(n, ph, m):
    e_per_w = m // NW
    n_iters = e_per_w // CH
    half_rows = n_iters // 2
    pairs = n_iters // 2
    assert e_per_w * NW == m and pairs * 2 * CH == e_per_w
    phw = ph // 2
    kg = phw // LANES

    scratch = [
        pltpu.VMEM((half_rows, CH), jnp.int32),
        pltpu.VMEM((half_rows, CH), jnp.int32),
        pltpu.VMEM((CH, phw), jnp.float32),
        pltpu.VMEM((CH, phw), jnp.float32),
        pltpu.VMEM((CH, phw), jnp.float32),
        pltpu.VMEM((CH, phw), jnp.float32),
        pltpu.VMEM((ph,), jnp.float32),
        pltpu.VMEM((LANES,), jnp.float32),
        pltpu.VMEM((CH,), jnp.float32),
        pltpu.SemaphoreType.DMA,
        pltpu.SemaphoreType.DMA,
        pltpu.SemaphoreType.DMA,
        pltpu.SemaphoreType.DMA,
    ]

    def body(sa_hbm, sb_hbm, u_hbm, v_hbm, w2_hbm, b2_hbm, out_hbm,
             u_v, v_v, sa0, sa1, sb0, sb1, w2_v, b2_v, sc_v,
             sa_sem0, sa_sem1, sb_sem0, sb_sem1):
        c = lax.axis_index("c")
        s = lax.axis_index("s")
        wid = c * NS + s
        sa_bufs = (sa0, sa1)
        sb_bufs = (sb0, sb1)
        sa_sems = (sa_sem0, sa_sem1)
        sb_sems = (sb_sem0, sb_sem1)

        pltpu.sync_copy(u_hbm.at[wid, 0], u_v)
        pltpu.sync_copy(v_hbm.at[wid, 0], v_v)
        pltpu.sync_copy(w2_hbm, w2_v)
        pltpu.sync_copy(b2_hbm, b2_v)
        w2lo = [w2_v[pl.ds(k * LANES, LANES)] for k in range(kg)]
        w2hi = [w2_v[pl.ds(phw + k * LANES, LANES)] for k in range(kg)]
        himask = jnp.full((LANES,), -65536, jnp.int32)
        b2r = b2_v[...]
        lane = lax.iota(jnp.int32, LANES)
        pidx = {mm: (lane ^ mm)[:, None] for mm in (8, 4, 2, 1)}
        m1 = (lane & 1) != 0
        m2 = (lane & 2) != 0
        bsel = [(lane >> 2) == b4 for b4 in range(4)]
        dn = lax.GatherDimensionNumbers(
            offset_dims=(), collapsed_slice_dims=(0,), start_index_map=(0,)
        )

        def perm(vv, mm):
            return lax.gather(
                vv, pidx[mm], dn, (1,),
                mode=lax.GatherScatterMode.PROMISE_IN_BOUNDS,
            )

        pltpu.async_copy(sa_hbm.at[u_v.at[0]], sa0, sa_sem0)
        pltpu.async_copy(sb_hbm.at[v_v.at[0]], sb0, sb_sem0)

        def half(j, b):
            @pl.when(j == half_rows - 1)
            def _():
                pltpu.sync_copy(u_hbm.at[wid, 1], u_v)
                pltpu.sync_copy(v_hbm.at[wid, 1], v_v)

            @pl.when(j + 1 < n_iters)
            def _():
                row = lax.rem(j + 1, half_rows)
                pltpu.async_copy(
                    sa_hbm.at[u_v.at[row]], sa_bufs[1 - b], sa_sems[1 - b]
                )
                pltpu.async_copy(
                    sb_hbm.at[v_v.at[row]], sb_bufs[1 - b], sb_sems[1 - b]
                )

            pltpu.make_async_copy(
                sa_hbm.at[pl.ds(0, CH)], sa_bufs[b], sa_sems[b]
            ).wait()
            pltpu.make_async_copy(
                sb_hbm.at[pl.ds(0, CH)], sb_bufs[b], sb_sems[b]
            ).wait()
            sa_v = sa_bufs[b]
            sb_v = sb_bufs[b]

            def grp(g, _):
                accs = [jnp.zeros((LANES,), jnp.float32)] * LANES
                for k in range(kg):
                    wlo = w2lo[k]
                    whi = w2hi[k]
                    for i in range(LANES):
                        ee = g * LANES + i
                        ia = lax.bitcast_convert_type(
                            sa_v[ee, pl.ds(k * LANES, LANES)], jnp.int32
                        )
                        ib = lax.bitcast_convert_type(
                            sb_v[ee, pl.ds(k * LANES, LANES)], jnp.int32
                        )
                        alo = lax.bitcast_convert_type(ia << 16, jnp.float32)
                        blo = lax.bitcast_convert_type(ib << 16, jnp.float32)
                        ahi = lax.bitcast_convert_type(ia & himask, jnp.float32)
                        bhi = lax.bitcast_convert_type(ib & himask, jnp.float32)
                        accs[i] = (
                            accs[i]
                            + jnp.maximum(alo + blo, 0.0) * wlo
                            + jnp.maximum(ahi + bhi, 0.0) * whi
                        )
                res = jnp.zeros((LANES,), jnp.float32)
                for b4 in range(4):
                    v0, v1, v2, v3 = accs[4 * b4 : 4 * b4 + 4]
                    c01 = jnp.where(m1, v1, v0) + perm(jnp.where(m1, v0, v1), 1)
                    c23 = jnp.where(m1, v3, v2) + perm(jnp.where(m1, v2, v3), 1)
                    cc = jnp.where(m2, c23, c01) + perm(jnp.where(m2, c01, c23), 2)
                    d = cc + perm(cc, 4)
                    e = d + perm(d, 8)
                    res = jnp.where(bsel[b4], e, res)
                sc_v[pl.ds(g * LANES, LANES)] = res + b2r
                return 0

            lax.fori_loop(0, CH // LANES, grp, 0)
            pltpu.sync_copy(
                sc_v, out_hbm.at[pl.ds(wid * e_per_w + j * CH, CH)]
            )

        def pair(jj, _):
            half(2 * jj, 0)
            half(2 * jj + 1, 1)
            return 0

        lax.fori_loop(0, pairs, pair, 0)

    return pl.kernel(
        body,
        out_type=jax.ShapeDtypeStruct((m,), jnp.float32),
        mesh=_mesh(),
        scratch_types=scratch,
    )


def _tc1_body(x_ref, p_ref, dp_ref, ws_ref, wn_ref, b_ref, h1_ref, deg_ref):
    deg = jnp.maximum(dp_ref[0, :, 0] + dp_ref[1, :, 0], 1.0)[:, None]
    agg = (p_ref[0] + p_ref[1]) / deg
    h = (
        jnp.dot(x_ref[...], ws_ref[...], preferred_element_type=jnp.float32)
        + jnp.dot(agg, wn_ref[...], preferred_element_type=jnp.float32)
        + b_ref[...]
    )
    h1_ref[...] = jnp.maximum(h, 0.0)
    deg_ref[...] = deg


def _tc2_body(x_ref, h1_ref, q_ref, deg_ref, ws_ref, wn_ref, b_ref,
              wp1_ref, bp1_ref, sa_ref, sb_ref):
    d = x_ref.shape[1]
    h = h1_ref.shape[1]
    agg = (q_ref[0] + q_ref[1]) / deg_ref[...]
    h2 = (
        jnp.dot(h1_ref[...], ws_ref[...], preferred_element_type=jnp.float32)
        + jnp.dot(agg, wn_ref[...], preferred_element_type=jnp.float32)
        + b_ref[...]
    )
    wp1 = wp1_ref[...]
    sa_ref[...] = (
        jnp.dot(x_ref[...], wp1[:d], preferred_element_type=jnp.float32)
        + bp1_ref[...]
    ).astype(jnp.bfloat16)
    sb_ref[...] = (
        jnp.dot(h2, wp1[d : d + h], preferred_element_type=jnp.float32)
        + jnp.dot(x_ref[...], wp1[d + h :], preferred_element_type=jnp.float32)
    ).astype(jnp.bfloat16)


def kernel(x, edge_index, pos_edge_index, neg_edge_index,
           Wself1, Wneigh1, bs1, Wself2, Wneigh2, bs2,
           Wp1, bp1, Wp2, bp2):
    n, d = x.shape
    h = Wself1.shape[1]
    ph = Wp1.shape[1]
    e = edge_index.shape[1]
    ep = pos_edge_index.shape[1]

    src4d = edge_index[0].reshape(NW, 2, -1, CHS)
    dst3d = edge_index[1].reshape(NW, -1, CHS)

    seg1 = _make_seg_sum(n, d, e)
    seg2 = _make_seg_sum(n, h, e)

    degp = _make_deg(n, e, d)(dst3d)
    agg1p = seg1(x, src4d, dst3d)

    h1, deg = pl.pallas_call(
        _tc1_body,
        out_shape=(
            jax.ShapeDtypeStruct((n, h), jnp.float32),
            jax.ShapeDtypeStruct((n, 1), jnp.float32),
        ),
    )(x, agg1p, degp, Wself1, Wneigh1, bs1.reshape(1, h))

    agg2p = seg2(h1, src4d, dst3d)

    sa, sb = pl.pallas_call(
        _tc2_body,
        out_shape=(
            jax.ShapeDtypeStruct((n, ph), jnp.bfloat16),
            jax.ShapeDtypeStruct((n, ph), jnp.bfloat16),
        ),
    )(x, h1, agg2p, deg, Wself2, Wneigh2, bs2.reshape(1, h),
      Wp1, bp1.reshape(1, ph))

    u4d = jnp.concatenate(
        [pos_edge_index[0], neg_edge_index[0]]
    ).reshape(NW, 2, -1, CH)
    v4d = jnp.concatenate(
        [pos_edge_index[1], neg_edge_index[1]]
    ).reshape(NW, 2, -1, CH)
    sap = lax.bitcast_convert_type(sa.reshape(n, ph // 2, 2), jnp.float32)
    sbp = lax.bitcast_convert_type(sb.reshape(n, ph // 2, 2), jnp.float32)
    w2 = Wp2[:, 0]
    w2p = jnp.concatenate([w2[0::2], w2[1::2]])
    b2v = jnp.full((LANES,), bp2[0], jnp.float32)

    scorer = _make_scorer(n, ph, 2 * ep)
    scores = scorer(sap, sbp, u4d, v4d, w2p, b2v)
    return scores[:ep], scores[ep:]
